# baseline (speedup 1.0000x reference)
import jax
import jax.numpy as jnp
from jax import lax
from jax.experimental import pallas as pl
from jax.experimental.pallas import tpu as pltpu
from jax.experimental.pallas import tpu_sc as plsc

_B, _T, _D = 4, 2048, 4096
_RT = 1
_NW = 32
_ROWS_PER_W = _T // _NW
_NT = _ROWS_PER_W // _RT
_U = 8
_NVEC = _RT * _D // 16


def _sc_add_kernel(x_hbm, pe_hbm, out_hbm, pe_buf, x_buf,
                   pe_sems, in_sems, out_sems):
    wid = lax.axis_index("s") * 2 + lax.axis_index("c")
    base = wid * _ROWS_PER_W

    def pe_copy(t, k):
        return pltpu.make_async_copy(
            pe_hbm.at[pl.ds(base + t * _RT, _RT)], pe_buf.at[k],
            pe_sems.at[k])

    def in_copy(t, b, k):
        return pltpu.make_async_copy(
            x_hbm.at[b, pl.ds(base + t * _RT, _RT)], x_buf.at[b, k],
            in_sems.at[b, k])

    def out_copy(t, b, k):
        return pltpu.make_async_copy(
            x_buf.at[b, k], out_hbm.at[b, pl.ds(base + t * _RT, _RT)],
            out_sems.at[b, k])

    pe_copy(0, 0).start()
    for b in range(_B):
        in_copy(0, b, 0).start()

    def outer(t2, _):
        for k in (0, 1):
            t = t2 * 2 + k
            kn = 1 - k
            pe_copy(t, k).wait()

            @pl.when(t + 1 < _NT)
            def _():
                pe_copy(t + 1, kn).start()

            for b in range(_B):
                in_copy(t, b, k).wait()

                @pl.when(t + 1 < _NT)
                def _(_b=b):
                    @pl.when(t >= 1)
                    def _():
                        out_copy(t - 1, _b, kn).wait()
                    in_copy(t + 1, _b, kn).start()

                @plsc.parallel_loop(0, _NVEC, unroll=_U)
                def _vec_body(g, _b=b, _k=k):
                    r = g // (_D // 16)
                    c = (g % (_D // 16)) * 16
                    pe_v = pe_buf[_k, r, pl.ds(c, 16)]
                    plsc.addupdate(x_buf.at[_b, _k, r, pl.ds(c, 16)], pe_v)

                out_copy(t, b, k).start()
        return 0

    lax.fori_loop(0, _NT // 2, outer, 0)

    k_last = (_NT - 1) % 2
    for b in range(_B):
        out_copy(_NT - 2, b, 1 - k_last).wait()
        out_copy(_NT - 1, b, k_last).wait()


def kernel(x, pe):
    B, T, D = x.shape
    mesh = plsc.VectorSubcoreMesh(core_axis_name="c", subcore_axis_name="s")
    f = pl.kernel(
        _sc_add_kernel,
        mesh=mesh,
        out_type=jax.ShapeDtypeStruct((B, T, D), x.dtype),
        scratch_types=[
            pltpu.VMEM((2, _RT, _D), jnp.float32),
            pltpu.VMEM((_B, 2, _RT, _D), jnp.float32),
            pltpu.SemaphoreType.DMA((2,)),
            pltpu.SemaphoreType.DMA((_B, 2)),
            pltpu.SemaphoreType.DMA((_B, 2)),
        ],
    )
    return f(x, pe[:T])


# speedup vs baseline: 1.0395x; 1.0395x over previous
import jax
import jax.numpy as jnp
from jax import lax
from jax.experimental import pallas as pl
from jax.experimental.pallas import tpu as pltpu
from jax.experimental.pallas import tpu_sc as plsc

_B, _T, _D = 4, 2048, 4096
_RT = 2
_NW = 32
_ROWS_PER_W = _T // _NW
_NT = _ROWS_PER_W // _RT
_U = 8
_NVEC = _RT * _D // 16


def _sc_add_kernel(x_hbm, pe_hbm, out_hbm, pe_buf, x_buf,
                   pe_sems, in_sems, out_sems):
    wid = lax.axis_index("s") * 2 + lax.axis_index("c")
    base = wid * _ROWS_PER_W

    def pe_copy(t, k):
        return pltpu.make_async_copy(
            pe_hbm.at[pl.ds(base + t * _RT, _RT)], pe_buf.at[k],
            pe_sems.at[k])

    def in_copy(t, b, k):
        return pltpu.make_async_copy(
            x_hbm.at[b, pl.ds(base + t * _RT, _RT)], x_buf.at[b, k],
            in_sems.at[b, k])

    def out_copy(t, b, k):
        return pltpu.make_async_copy(
            x_buf.at[b, k], out_hbm.at[b, pl.ds(base + t * _RT, _RT)],
            out_sems.at[b, k])

    pe_copy(0, 0).start()
    for b in range(_B):
        in_copy(0, b, 0).start()

    def outer(t2, _):
        for k in (0, 1):
            t = t2 * 2 + k
            kn = 1 - k
            pe_copy(t, k).wait()

            @pl.when(t + 1 < _NT)
            def _():
                pe_copy(t + 1, kn).start()

            for b in range(_B):
                in_copy(t, b, k).wait()

                @pl.when(t + 1 < _NT)
                def _(_b=b):
                    @pl.when(t >= 1)
                    def _():
                        out_copy(t - 1, _b, kn).wait()
                    in_copy(t + 1, _b, kn).start()

                @plsc.parallel_loop(0, _NVEC, unroll=_U)
                def _vec_body(g, _b=b, _k=k):
                    r = g // (_D // 16)
                    c = (g % (_D // 16)) * 16
                    pe_v = pe_buf[_k, r, pl.ds(c, 16)]
                    plsc.addupdate(x_buf.at[_b, _k, r, pl.ds(c, 16)], pe_v)

                out_copy(t, b, k).start()
        return 0

    lax.fori_loop(0, _NT // 2, outer, 0)

    k_last = (_NT - 1) % 2
    for b in range(_B):
        out_copy(_NT - 2, b, 1 - k_last).wait()
        out_copy(_NT - 1, b, k_last).wait()


def kernel(x, pe):
    B, T, D = x.shape
    mesh = plsc.VectorSubcoreMesh(core_axis_name="c", subcore_axis_name="s")
    f = pl.kernel(
        _sc_add_kernel,
        mesh=mesh,
        out_type=jax.ShapeDtypeStruct((B, T, D), x.dtype),
        scratch_types=[
            pltpu.VMEM((2, _RT, _D), jnp.float32),
            pltpu.VMEM((_B, 2, _RT, _D), jnp.float32),
            pltpu.SemaphoreType.DMA((2,)),
            pltpu.SemaphoreType.DMA((_B, 2)),
            pltpu.SemaphoreType.DMA((_B, 2)),
        ],
    )
    return f(x, pe[:T])
